# Initial kernel scaffold; baseline (speedup 1.0000x reference)
import functools

import jax
import jax.numpy as jnp
from jax import lax
from jax.experimental import pallas as pl
from jax.experimental.pallas import tpu as pltpu
from jax.experimental.pallas import tpu_sc as plsc

N_NODES = 10000
D = 128
EXPERTS = 3
TEMP = 101.0

NC = 2
NS = 16
NW = NC * NS
L = 16

CHUNK = 128
NPAD = 10240
RPT = NPAD // NS

_mesh = plsc.VectorSubcoreMesh(
    core_axis_name="c", subcore_axis_name="s", num_cores=NC, num_subcores=NS
)


def _deg_call(dst_p, cpt):
    @functools.partial(
        pl.kernel,
        mesh=_mesh,
        out_type=jax.ShapeDtypeStruct((NC, NPAD), jnp.float32),
        scratch_types=[
            pltpu.VMEM((cpt, CHUNK), jnp.int32),
            pltpu.VMEM((CHUNK,), jnp.float32),
            pltpu.VMEM((RPT,), jnp.float32),
            pltpu.VMEM_SHARED((NPAD,), jnp.float32),
        ],
    )
    def k(dst_hbm, out_hbm, dstv, ones_v, zbuf, deg_sh):
        cid = lax.axis_index("c")
        sid = lax.axis_index("s")
        wid = sid * NC + cid

        def fill_ones(i, carry):
            ones_v[pl.ds(i * L, L)] = jnp.ones((L,), jnp.float32)
            return carry

        lax.fori_loop(0, CHUNK // L, fill_ones, 0)

        def fill_zero(i, carry):
            zbuf[pl.ds(i * L, L)] = jnp.zeros((L,), jnp.float32)
            return carry

        lax.fori_loop(0, RPT // L, fill_zero, 0)

        pltpu.sync_copy(zbuf, deg_sh.at[pl.ds(sid * RPT, RPT)])
        pltpu.sync_copy(dst_hbm.at[wid], dstv)
        plsc.subcore_barrier()

        def body(c, carry):
            pltpu.sync_copy(ones_v, deg_sh.at[dstv.at[c]], add=True)
            return carry

        lax.fori_loop(0, cpt, body, 0)
        plsc.subcore_barrier()
        pltpu.sync_copy(
            deg_sh.at[pl.ds(sid * RPT, RPT)],
            out_hbm.at[cid, pl.ds(sid * RPT, RPT)],
        )

    return k(dst_p)


def _agg_call(xs, src_p, dst_p, zeros_big, cpt):
    @functools.partial(
        pl.kernel,
        mesh=_mesh,
        out_type=jax.ShapeDtypeStruct((NC, NPAD, D), jnp.float32),
        scratch_types=[
            pltpu.VMEM((cpt, CHUNK), jnp.int32),
            pltpu.VMEM((cpt, CHUNK), jnp.int32),
            pltpu.VMEM((CHUNK, D), jnp.float32),
            pltpu.VMEM_SHARED((NPAD, D), jnp.float32),
            pltpu.SemaphoreType.DMA,
        ],
    )
    def k(xs_hbm, src_hbm, dst_hbm, zeros_hbm, out_hbm, srcv, dstv, rows, agg_sh, sem):
        cid = lax.axis_index("c")
        sid = lax.axis_index("s")
        wid = sid * NC + cid

        pltpu.sync_copy(
            zeros_hbm.at[pl.ds(sid * RPT, RPT)],
            agg_sh.at[pl.ds(sid * RPT, RPT)],
        )
        pltpu.sync_copy(src_hbm.at[wid], srcv)
        pltpu.sync_copy(dst_hbm.at[wid], dstv)
        plsc.subcore_barrier()

        def body(c, carry):
            pltpu.async_copy(xs_hbm.at[srcv.at[c]], rows, sem).wait()
            pltpu.sync_copy(rows, agg_sh.at[dstv.at[c]], add=True)
            return carry

        lax.fori_loop(0, cpt, body, 0)
        plsc.subcore_barrier()
        pltpu.sync_copy(
            agg_sh.at[pl.ds(sid * RPT, RPT)],
            out_hbm.at[cid, pl.ds(sid * RPT, RPT)],
        )

    return k(xs, src_p, dst_p, zeros_big)


def _prep_call(deg_parts, x_pad, gf_pad, Wg):
    def body(degp_ref, x_ref, gf_ref, wg_ref, xs_ref, dsi_ref, gate_ref):
        deg = degp_ref[0] + degp_ref[1] + 1.0
        dsi = lax.rsqrt(deg)
        xs_ref[...] = x_ref[...] * dsi[:, None]
        dsi_ref[...] = dsi[:, None]
        logits = jnp.dot(gf_ref[...], wg_ref[...], preferred_element_type=jnp.float32)
        logits = logits * (1.0 / TEMP)
        m = jnp.max(logits, axis=-1, keepdims=True)
        e = jnp.exp(logits - m)
        gate_ref[...] = e / jnp.sum(e, axis=-1, keepdims=True)

    return pl.pallas_call(
        body,
        out_shape=(
            jax.ShapeDtypeStruct((NPAD, D), jnp.float32),
            jax.ShapeDtypeStruct((NPAD, 1), jnp.float32),
            jax.ShapeDtypeStruct((NPAD, EXPERTS), jnp.float32),
        ),
    )(deg_parts, x_pad, gf_pad, Wg)


def _combine_call(agg_parts, xs, dsi, gate, W, b):
    BR = 1280

    def body(a_ref, xs_ref, dsi_ref, gate_ref, w_ref, b_ref, o_ref):
        agg = (a_ref[0] + a_ref[1] + xs_ref[...]) * dsi_ref[...]
        acc = jnp.zeros((BR, D), jnp.float32)
        for i in range(EXPERTS):
            h = jnp.dot(agg, w_ref[i], preferred_element_type=jnp.float32)
            h = h + b_ref[i][None, :]
            acc = acc + gate_ref[:, i][:, None] * jnp.maximum(h, 0.0)
        o_ref[...] = acc

    return pl.pallas_call(
        body,
        grid=(NPAD // BR,),
        in_specs=[
            pl.BlockSpec((NC, BR, D), lambda i: (0, i, 0)),
            pl.BlockSpec((BR, D), lambda i: (i, 0)),
            pl.BlockSpec((BR, 1), lambda i: (i, 0)),
            pl.BlockSpec((BR, EXPERTS), lambda i: (i, 0)),
            pl.BlockSpec((EXPERTS, D, D), lambda i: (0, 0, 0)),
            pl.BlockSpec((EXPERTS, D), lambda i: (0, 0)),
        ],
        out_specs=pl.BlockSpec((BR, D), lambda i: (i, 0)),
        out_shape=jax.ShapeDtypeStruct((NPAD, D), jnp.float32),
    )(agg_parts, xs, dsi, gate, W, b)


def kernel(x, edge_index, gate_features, W, b, Wg):
    ei = edge_index.astype(jnp.int32)
    src, dst = ei[0], ei[1]
    e = src.shape[0]
    cpt = -(-e // (NW * CHUNK))
    pad = NW * cpt * CHUNK - e
    src_p = jnp.concatenate(
        [src, jnp.full((pad,), N_NODES, jnp.int32)]
    ).reshape(NW, cpt, CHUNK)
    dst_p = jnp.concatenate(
        [dst, jnp.full((pad,), N_NODES, jnp.int32)]
    ).reshape(NW, cpt, CHUNK)

    x_pad = jnp.pad(x.astype(jnp.float32), ((0, NPAD - N_NODES), (0, 0)))
    gf_pad = jnp.pad(gate_features.astype(jnp.float32), ((0, NPAD - N_NODES), (0, 0)))

    deg_parts = _deg_call(dst_p, cpt)
    xs, dsi, gate = _prep_call(deg_parts, x_pad, gf_pad, Wg)
    zeros_big = jnp.zeros((NPAD, D), jnp.float32)
    agg_parts = _agg_call(xs, src_p, dst_p, zeros_big, cpt)
    out = _combine_call(agg_parts, xs, dsi, gate, W, b)
    return out[:N_NODES]


# speedup vs baseline: 32.0081x; 32.0081x over previous
import functools

import jax
import jax.numpy as jnp
from jax import lax
from jax.experimental import pallas as pl
from jax.experimental.pallas import tpu as pltpu
from jax.experimental.pallas import tpu_sc as plsc

N_NODES = 10000
D = 128
EXPERTS = 3
TEMP = 101.0

NC = 2
NS = 16
NW = NC * NS
L = 16

CHUNK = 128
NPAD = 10240
RPT = NPAD // NS

def _mesh():
    return plsc.VectorSubcoreMesh(
        core_axis_name="c", subcore_axis_name="s", num_cores=NC, num_subcores=NS
    )


def _deg_call(dst_p, cpt):
    @functools.partial(
        pl.kernel,
        mesh=_mesh(),
        out_type=jax.ShapeDtypeStruct((NC, NPAD), jnp.float32),
        scratch_types=[
            pltpu.VMEM((cpt, CHUNK), jnp.int32),
            pltpu.VMEM((CHUNK,), jnp.float32),
            pltpu.VMEM((RPT,), jnp.float32),
            pltpu.VMEM_SHARED((NPAD,), jnp.float32),
        ],
    )
    def k(dst_hbm, out_hbm, dstv, ones_v, zbuf, deg_sh):
        cid = lax.axis_index("c")
        sid = lax.axis_index("s")
        wid = sid * NC + cid

        def fill_ones(i, carry):
            ones_v[pl.ds(i * L, L)] = jnp.ones((L,), jnp.float32)
            return carry

        lax.fori_loop(0, CHUNK // L, fill_ones, 0)

        def fill_zero(i, carry):
            zbuf[pl.ds(i * L, L)] = jnp.zeros((L,), jnp.float32)
            return carry

        lax.fori_loop(0, RPT // L, fill_zero, 0)

        pltpu.sync_copy(zbuf, deg_sh.at[pl.ds(sid * RPT, RPT)])
        pltpu.sync_copy(dst_hbm.at[wid], dstv)
        plsc.subcore_barrier()

        def body(c, carry):
            pltpu.sync_copy(ones_v, deg_sh.at[dstv.at[c]], add=True)
            return carry

        lax.fori_loop(0, cpt, body, 0)
        plsc.subcore_barrier()
        pltpu.sync_copy(
            deg_sh.at[pl.ds(sid * RPT, RPT)],
            out_hbm.at[cid, pl.ds(sid * RPT, RPT)],
        )

    return k(dst_p)


def _agg_call(xs, src_p, dst_p, zeros_big, cpt):
    @functools.partial(
        pl.kernel,
        mesh=_mesh(),
        out_type=jax.ShapeDtypeStruct((NC, NPAD, D), jnp.float32),
        scratch_types=[
            pltpu.VMEM((cpt, CHUNK), jnp.int32),
            pltpu.VMEM((cpt, CHUNK), jnp.int32),
            pltpu.VMEM((CHUNK, D), jnp.float32),
            pltpu.VMEM_SHARED((NPAD, D), jnp.float32),
            pltpu.SemaphoreType.DMA,
        ],
    )
    def k(xs_hbm, src_hbm, dst_hbm, zeros_hbm, out_hbm, srcv, dstv, rows, agg_sh, sem):
        cid = lax.axis_index("c")
        sid = lax.axis_index("s")
        wid = sid * NC + cid

        pltpu.sync_copy(
            zeros_hbm.at[pl.ds(sid * RPT, RPT)],
            agg_sh.at[pl.ds(sid * RPT, RPT)],
        )
        pltpu.sync_copy(src_hbm.at[wid], srcv)
        pltpu.sync_copy(dst_hbm.at[wid], dstv)
        plsc.subcore_barrier()

        def body(c, carry):
            pltpu.async_copy(xs_hbm.at[srcv.at[c]], rows, sem).wait()
            pltpu.sync_copy(rows, agg_sh.at[dstv.at[c]], add=True)
            return carry

        lax.fori_loop(0, cpt, body, 0)
        plsc.subcore_barrier()
        pltpu.sync_copy(
            agg_sh.at[pl.ds(sid * RPT, RPT)],
            out_hbm.at[cid, pl.ds(sid * RPT, RPT)],
        )

    return k(xs, src_p, dst_p, zeros_big)


def _prep_call(deg_parts, x_pad, gf_pad, Wg):
    def body(degp_ref, x_ref, gf_ref, wg_ref, xs_ref, dsi_ref, gate_ref):
        deg = degp_ref[0] + degp_ref[1] + 1.0
        dsi = lax.rsqrt(deg)
        xs_ref[...] = x_ref[...] * dsi[:, None]
        dsi_ref[...] = dsi[:, None]
        logits = jnp.dot(gf_ref[...], wg_ref[...], preferred_element_type=jnp.float32)
        logits = logits * (1.0 / TEMP)
        m = jnp.max(logits, axis=-1, keepdims=True)
        e = jnp.exp(logits - m)
        gate_ref[...] = e / jnp.sum(e, axis=-1, keepdims=True)

    return pl.pallas_call(
        body,
        out_shape=(
            jax.ShapeDtypeStruct((NPAD, D), jnp.float32),
            jax.ShapeDtypeStruct((NPAD, 1), jnp.float32),
            jax.ShapeDtypeStruct((NPAD, EXPERTS), jnp.float32),
        ),
    )(deg_parts, x_pad, gf_pad, Wg)


def _combine_call(agg_parts, xs, dsi, gate, W, b):
    BR = 1280

    def body(a_ref, xs_ref, dsi_ref, gate_ref, w_ref, b_ref, o_ref):
        agg = (a_ref[0] + a_ref[1] + xs_ref[...]) * dsi_ref[...]
        acc = jnp.zeros((BR, D), jnp.float32)
        for i in range(EXPERTS):
            h = jnp.dot(agg, w_ref[i], preferred_element_type=jnp.float32)
            h = h + b_ref[i][None, :]
            acc = acc + gate_ref[:, i][:, None] * jnp.maximum(h, 0.0)
        o_ref[...] = acc

    return pl.pallas_call(
        body,
        grid=(NPAD // BR,),
        in_specs=[
            pl.BlockSpec((NC, BR, D), lambda i: (0, i, 0)),
            pl.BlockSpec((BR, D), lambda i: (i, 0)),
            pl.BlockSpec((BR, 1), lambda i: (i, 0)),
            pl.BlockSpec((BR, EXPERTS), lambda i: (i, 0)),
            pl.BlockSpec((EXPERTS, D, D), lambda i: (0, 0, 0)),
            pl.BlockSpec((EXPERTS, D), lambda i: (0, 0)),
        ],
        out_specs=pl.BlockSpec((BR, D), lambda i: (i, 0)),
        out_shape=jax.ShapeDtypeStruct((NPAD, D), jnp.float32),
    )(agg_parts, xs, dsi, gate, W, b)


def kernel(x, edge_index, gate_features, W, b, Wg):
    ei = edge_index.astype(jnp.int32)
    src, dst = ei[0], ei[1]
    e = src.shape[0]
    cpt = -(-e // (NW * CHUNK))
    pad = NW * cpt * CHUNK - e
    src_p = jnp.concatenate(
        [src, jnp.full((pad,), N_NODES, jnp.int32)]
    ).reshape(NW, cpt, CHUNK)
    dst_p = jnp.concatenate(
        [dst, jnp.full((pad,), N_NODES, jnp.int32)]
    ).reshape(NW, cpt, CHUNK)

    x_pad = jnp.pad(x.astype(jnp.float32), ((0, NPAD - N_NODES), (0, 0)))
    gf_pad = jnp.pad(gate_features.astype(jnp.float32), ((0, NPAD - N_NODES), (0, 0)))

    deg_parts = _deg_call(dst_p, cpt)
    xs, dsi, gate = _prep_call(deg_parts, x_pad, gf_pad, Wg)
    zeros_big = jnp.zeros((NPAD, D), jnp.float32)
    agg_parts = _agg_call(xs, src_p, dst_p, zeros_big, cpt)
    out = _combine_call(agg_parts, xs, dsi, gate, W, b)
    return out[:N_NODES]


# speedup vs baseline: 65.8023x; 2.0558x over previous
import functools

import jax
import jax.numpy as jnp
from jax import lax
from jax.experimental import pallas as pl
from jax.experimental.pallas import tpu as pltpu
from jax.experimental.pallas import tpu_sc as plsc

N_NODES = 10000
D = 128
EXPERTS = 3
TEMP = 101.0

NC = 2
NS = 16
NW = NC * NS
L = 16

CHUNK = 128
RING = 4
NPAD = 10240
RPT = NPAD // NS


def _mesh():
    return plsc.VectorSubcoreMesh(
        core_axis_name="c", subcore_axis_name="s", num_cores=NC, num_subcores=NS
    )


def _deg_call(sd, cpt):
    @functools.partial(
        pl.kernel,
        mesh=_mesh(),
        out_type=jax.ShapeDtypeStruct((NC, NPAD), jnp.float32),
        scratch_types=[
            pltpu.VMEM((cpt, 2, CHUNK), jnp.int32),
            pltpu.VMEM((CHUNK,), jnp.float32),
            pltpu.VMEM((RPT,), jnp.float32),
            pltpu.VMEM_SHARED((NPAD,), jnp.float32),
        ],
    )
    def k(sd_hbm, out_hbm, idxv, ones_v, zbuf, deg_sh):
        cid = lax.axis_index("c")
        sid = lax.axis_index("s")
        wid = sid * NC + cid

        def fill_ones(i, carry):
            ones_v[pl.ds(i * L, L)] = jnp.ones((L,), jnp.float32)
            return carry

        lax.fori_loop(0, CHUNK // L, fill_ones, 0)

        def fill_zero(i, carry):
            zbuf[pl.ds(i * L, L)] = jnp.zeros((L,), jnp.float32)
            return carry

        lax.fori_loop(0, RPT // L, fill_zero, 0)

        pltpu.sync_copy(zbuf, deg_sh.at[pl.ds(sid * RPT, RPT)])
        pltpu.sync_copy(sd_hbm.at[wid], idxv)
        plsc.subcore_barrier()

        def body(c, carry):
            pltpu.sync_copy(ones_v, deg_sh.at[idxv.at[c, 1]], add=True)
            return carry

        lax.fori_loop(0, cpt, body, 0)
        plsc.subcore_barrier()
        pltpu.sync_copy(
            deg_sh.at[pl.ds(sid * RPT, RPT)],
            out_hbm.at[cid, pl.ds(sid * RPT, RPT)],
        )

    return k(sd)


def _agg_call(xs, sd, zeros_big, cpt):
    @functools.partial(
        pl.kernel,
        mesh=_mesh(),
        out_type=jax.ShapeDtypeStruct((NC, NPAD, D), jnp.float32),
        scratch_types=[
            pltpu.VMEM((RING, 2, CHUNK), jnp.int32),
            pltpu.VMEM((CHUNK, D), jnp.float32),
            pltpu.VMEM((CHUNK, D), jnp.float32),
            pltpu.VMEM_SHARED((NPAD, D), jnp.float32),
            pltpu.SemaphoreType.DMA,
            pltpu.SemaphoreType.DMA,
            pltpu.SemaphoreType.DMA,
            pltpu.SemaphoreType.DMA,
        ],
    )
    def k(xs_hbm, sd_hbm, zeros_hbm, out_hbm, idxv, ra, rb, agg_sh, sie, sio, sa, sb):
        cid = lax.axis_index("c")
        sid = lax.axis_index("s")
        wid = sid * NC + cid

        pltpu.sync_copy(
            zeros_hbm.at[pl.ds(sid * RPT, RPT)],
            agg_sh.at[pl.ds(sid * RPT, RPT)],
        )
        pltpu.async_copy(sd_hbm.at[wid, 0], idxv.at[0], sie)
        pltpu.make_async_copy(sd_hbm.at[wid, 0], idxv.at[0], sie).wait()
        pltpu.async_copy(sd_hbm.at[wid, 1], idxv.at[1], sio)
        pltpu.async_copy(sd_hbm.at[wid, 2], idxv.at[2], sie)
        plsc.subcore_barrier()
        pltpu.async_copy(xs_hbm.at[idxv.at[0, 0]], ra, sa)

        def body(i, carry):
            c = 2 * i
            r0 = lax.rem(c, RING)
            r1 = lax.rem(c + 1, RING)
            r2 = lax.rem(c + 2, RING)
            r3 = lax.rem(c + 3, RING)
            pltpu.make_async_copy(sd_hbm.at[wid, 0], idxv.at[0], sio).wait()
            pltpu.async_copy(xs_hbm.at[idxv.at[r1, 0]], rb, sb)

            @pl.when(c + 3 < cpt)
            def _():
                pltpu.async_copy(sd_hbm.at[wid, c + 3], idxv.at[r3], sio)

            pltpu.make_async_copy(xs_hbm.at[idxv.at[r0, 0]], ra, sa).wait()
            pltpu.sync_copy(ra, agg_sh.at[idxv.at[r0, 1]], add=True)

            @pl.when(c + 2 < cpt)
            def _():
                pltpu.make_async_copy(sd_hbm.at[wid, 0], idxv.at[0], sie).wait()
                pltpu.async_copy(xs_hbm.at[idxv.at[r2, 0]], ra, sa)

                @pl.when(c + 4 < cpt)
                def _():
                    pltpu.async_copy(
                        sd_hbm.at[wid, c + 4], idxv.at[lax.rem(c + 4, RING)], sie
                    )

            pltpu.make_async_copy(xs_hbm.at[idxv.at[r1, 0]], rb, sb).wait()
            pltpu.sync_copy(rb, agg_sh.at[idxv.at[r1, 1]], add=True)
            return carry

        lax.fori_loop(0, cpt // 2, body, 0)
        plsc.subcore_barrier()
        pltpu.sync_copy(
            agg_sh.at[pl.ds(sid * RPT, RPT)],
            out_hbm.at[cid, pl.ds(sid * RPT, RPT)],
        )

    return k(xs, sd, zeros_big)


def _prep_call(deg_parts, x_pad, gf_pad, Wg):
    def body(degp_ref, x_ref, gf_ref, wg_ref, xs_ref, dsi_ref, gate_ref):
        deg = degp_ref[0] + degp_ref[1] + 1.0
        dsi = lax.rsqrt(deg)
        xs_ref[...] = x_ref[...] * dsi[:, None]
        dsi_ref[...] = dsi[:, None]
        logits = jnp.dot(gf_ref[...], wg_ref[...], preferred_element_type=jnp.float32)
        logits = logits * (1.0 / TEMP)
        m = jnp.max(logits, axis=-1, keepdims=True)
        e = jnp.exp(logits - m)
        gate_ref[...] = e / jnp.sum(e, axis=-1, keepdims=True)

    return pl.pallas_call(
        body,
        out_shape=(
            jax.ShapeDtypeStruct((NPAD, D), jnp.float32),
            jax.ShapeDtypeStruct((NPAD, 1), jnp.float32),
            jax.ShapeDtypeStruct((NPAD, EXPERTS), jnp.float32),
        ),
    )(deg_parts, x_pad, gf_pad, Wg)


def _combine_call(agg_parts, xs, dsi, gate, W, b):
    BR = 1280

    def body(a_ref, xs_ref, dsi_ref, gate_ref, w_ref, b_ref, o_ref):
        agg = (a_ref[0] + a_ref[1] + xs_ref[...]) * dsi_ref[...]
        acc = jnp.zeros((BR, D), jnp.float32)
        for i in range(EXPERTS):
            h = jnp.dot(agg, w_ref[i], preferred_element_type=jnp.float32)
            h = h + b_ref[i][None, :]
            acc = acc + gate_ref[:, i][:, None] * jnp.maximum(h, 0.0)
        o_ref[...] = acc

    return pl.pallas_call(
        body,
        grid=(NPAD // BR,),
        in_specs=[
            pl.BlockSpec((NC, BR, D), lambda i: (0, i, 0)),
            pl.BlockSpec((BR, D), lambda i: (i, 0)),
            pl.BlockSpec((BR, 1), lambda i: (i, 0)),
            pl.BlockSpec((BR, EXPERTS), lambda i: (i, 0)),
            pl.BlockSpec((EXPERTS, D, D), lambda i: (0, 0, 0)),
            pl.BlockSpec((EXPERTS, D), lambda i: (0, 0)),
        ],
        out_specs=pl.BlockSpec((BR, D), lambda i: (i, 0)),
        out_shape=jax.ShapeDtypeStruct((NPAD, D), jnp.float32),
    )(agg_parts, xs, dsi, gate, W, b)


def kernel(x, edge_index, gate_features, W, b, Wg):
    ei = edge_index.astype(jnp.int32)
    src, dst = ei[0], ei[1]
    e = src.shape[0]
    cpt = -(-e // (NW * CHUNK))
    cpt = cpt + (cpt % 2)
    pad = NW * cpt * CHUNK - e
    pad_rows = N_NODES + (jnp.arange(pad, dtype=jnp.int32) % (NPAD - N_NODES))
    src_p = jnp.concatenate([src, pad_rows]).reshape(NW, cpt, CHUNK)
    dst_p = jnp.concatenate([dst, pad_rows]).reshape(NW, cpt, CHUNK)
    sd = jnp.stack([src_p, dst_p], axis=2)

    x_pad = jnp.pad(x.astype(jnp.float32), ((0, NPAD - N_NODES), (0, 0)))
    gf_pad = jnp.pad(gate_features.astype(jnp.float32), ((0, NPAD - N_NODES), (0, 0)))

    deg_parts = _deg_call(sd, cpt)
    xs, dsi, gate = _prep_call(deg_parts, x_pad, gf_pad, Wg)
    zeros_big = jnp.zeros((NPAD, D), jnp.float32)
    agg_parts = _agg_call(xs, sd, zeros_big, cpt)
    out = _combine_call(agg_parts, xs, dsi, gate, W, b)
    return out[:N_NODES]


# speedup vs baseline: 66.3265x; 1.0080x over previous
import functools

import jax
import jax.numpy as jnp
from jax import lax
from jax.experimental import pallas as pl
from jax.experimental.pallas import tpu as pltpu
from jax.experimental.pallas import tpu_sc as plsc

N_NODES = 10000
D = 128
EXPERTS = 3
TEMP = 101.0

NC = 2
NS = 16
NW = NC * NS
L = 16

CHUNK = 128
RING = 4
NPAD = 10240
RPT = NPAD // NS


def _mesh():
    return plsc.VectorSubcoreMesh(
        core_axis_name="c", subcore_axis_name="s", num_cores=NC, num_subcores=NS
    )


def _deg_call(sd, cpt):
    @functools.partial(
        pl.kernel,
        mesh=_mesh(),
        out_type=jax.ShapeDtypeStruct((NC, NPAD), jnp.float32),
        scratch_types=[
            pltpu.VMEM((cpt, 2, CHUNK), jnp.int32),
            pltpu.VMEM((CHUNK,), jnp.float32),
            pltpu.VMEM((RPT,), jnp.float32),
            pltpu.VMEM_SHARED((NPAD,), jnp.float32),
        ],
    )
    def k(sd_hbm, out_hbm, idxv, ones_v, zbuf, deg_sh):
        cid = lax.axis_index("c")
        sid = lax.axis_index("s")
        wid = sid * NC + cid

        def fill_ones(i, carry):
            ones_v[pl.ds(i * L, L)] = jnp.ones((L,), jnp.float32)
            return carry

        lax.fori_loop(0, CHUNK // L, fill_ones, 0)

        def fill_zero(i, carry):
            zbuf[pl.ds(i * L, L)] = jnp.zeros((L,), jnp.float32)
            return carry

        lax.fori_loop(0, RPT // L, fill_zero, 0)

        pltpu.sync_copy(zbuf, deg_sh.at[pl.ds(sid * RPT, RPT)])
        pltpu.sync_copy(sd_hbm.at[wid], idxv)
        plsc.subcore_barrier()

        def body(c, carry):
            pltpu.sync_copy(ones_v, deg_sh.at[idxv.at[c, 1]], add=True)
            return carry

        lax.fori_loop(0, cpt, body, 0)
        plsc.subcore_barrier()
        pltpu.sync_copy(
            deg_sh.at[pl.ds(sid * RPT, RPT)],
            out_hbm.at[cid, pl.ds(sid * RPT, RPT)],
        )

    return k(sd)


def _agg_call(xs, sd, cpt):
    @functools.partial(
        pl.kernel,
        mesh=_mesh(),
        out_type=jax.ShapeDtypeStruct((NC, NPAD, D), jnp.float32),
        scratch_types=[
            pltpu.VMEM((RING, 2, CHUNK), jnp.int32),
            pltpu.VMEM((CHUNK, D), jnp.float32),
            pltpu.VMEM((CHUNK, D), jnp.float32),
            pltpu.VMEM_SHARED((NPAD, D), jnp.float32),
            pltpu.SemaphoreType.DMA,
            pltpu.SemaphoreType.DMA,
            pltpu.SemaphoreType.DMA,
            pltpu.SemaphoreType.DMA,
        ],
    )
    def k(xs_hbm, sd_hbm, out_hbm, idxv, ra, rb, agg_sh, sie, sio, sa, sb):
        cid = lax.axis_index("c")
        sid = lax.axis_index("s")
        wid = sid * NC + cid

        def zfill(i, carry):
            rb[lax.div(i, D // L), pl.ds(lax.rem(i, D // L) * L, L)] = jnp.zeros(
                (L,), jnp.float32
            )
            return carry

        lax.fori_loop(0, CHUNK * (D // L), zfill, 0)
        for kk in range(RPT // CHUNK):
            pltpu.sync_copy(rb, agg_sh.at[pl.ds(sid * RPT + kk * CHUNK, CHUNK)])
        pltpu.async_copy(sd_hbm.at[wid, 0], idxv.at[0], sie)
        pltpu.make_async_copy(sd_hbm.at[wid, 0], idxv.at[0], sie).wait()
        pltpu.async_copy(sd_hbm.at[wid, 1], idxv.at[1], sio)
        pltpu.async_copy(sd_hbm.at[wid, 2], idxv.at[2], sie)
        plsc.subcore_barrier()
        pltpu.async_copy(xs_hbm.at[idxv.at[0, 0]], ra, sa)

        def body(i, carry):
            c = 2 * i
            r0 = lax.rem(c, RING)
            r1 = lax.rem(c + 1, RING)
            r2 = lax.rem(c + 2, RING)
            r3 = lax.rem(c + 3, RING)
            pltpu.make_async_copy(sd_hbm.at[wid, 0], idxv.at[0], sio).wait()
            pltpu.async_copy(xs_hbm.at[idxv.at[r1, 0]], rb, sb)

            @pl.when(c + 3 < cpt)
            def _():
                pltpu.async_copy(sd_hbm.at[wid, c + 3], idxv.at[r3], sio)

            pltpu.make_async_copy(xs_hbm.at[idxv.at[r0, 0]], ra, sa).wait()
            pltpu.sync_copy(ra, agg_sh.at[idxv.at[r0, 1]], add=True)

            @pl.when(c + 2 < cpt)
            def _():
                pltpu.make_async_copy(sd_hbm.at[wid, 0], idxv.at[0], sie).wait()
                pltpu.async_copy(xs_hbm.at[idxv.at[r2, 0]], ra, sa)

                @pl.when(c + 4 < cpt)
                def _():
                    pltpu.async_copy(
                        sd_hbm.at[wid, c + 4], idxv.at[lax.rem(c + 4, RING)], sie
                    )

            pltpu.make_async_copy(xs_hbm.at[idxv.at[r1, 0]], rb, sb).wait()
            pltpu.sync_copy(rb, agg_sh.at[idxv.at[r1, 1]], add=True)
            return carry

        lax.fori_loop(0, cpt // 2, body, 0)
        plsc.subcore_barrier()
        pltpu.sync_copy(
            agg_sh.at[pl.ds(sid * RPT, RPT)],
            out_hbm.at[cid, pl.ds(sid * RPT, RPT)],
        )

    return k(xs, sd)


def _prep_call(deg_parts, x, gf, Wg):
    def body(degp_ref, x_ref, gf_ref, wg_ref, xs_ref, dsi_ref, gate_ref):
        deg = degp_ref[0] + degp_ref[1] + 1.0
        dsi = lax.rsqrt(deg)
        xs_ref[0:N_NODES] = x_ref[...] * dsi[0:N_NODES, None]
        xs_ref[N_NODES:NPAD] = jnp.zeros((NPAD - N_NODES, D), jnp.float32)
        dsi_ref[...] = dsi[:, None]
        logits = jnp.dot(gf_ref[...], wg_ref[...], preferred_element_type=jnp.float32)
        logits = logits * (1.0 / TEMP)
        m = jnp.max(logits, axis=-1, keepdims=True)
        e = jnp.exp(logits - m)
        gate_ref[0:N_NODES] = e / jnp.sum(e, axis=-1, keepdims=True)
        gate_ref[N_NODES:NPAD] = jnp.zeros((NPAD - N_NODES, EXPERTS), jnp.float32)

    return pl.pallas_call(
        body,
        out_shape=(
            jax.ShapeDtypeStruct((NPAD, D), jnp.float32),
            jax.ShapeDtypeStruct((NPAD, 1), jnp.float32),
            jax.ShapeDtypeStruct((NPAD, EXPERTS), jnp.float32),
        ),
    )(deg_parts, x, gf, Wg)


def _combine_call(agg_parts, xs, dsi, gate, W, b):
    BR = 1280

    def body(a_ref, xs_ref, dsi_ref, gate_ref, w_ref, b_ref, o_ref):
        agg = (a_ref[0] + a_ref[1] + xs_ref[...]) * dsi_ref[...]
        acc = jnp.zeros((BR, D), jnp.float32)
        for i in range(EXPERTS):
            h = jnp.dot(agg, w_ref[i], preferred_element_type=jnp.float32)
            h = h + b_ref[i][None, :]
            acc = acc + gate_ref[:, i][:, None] * jnp.maximum(h, 0.0)
        o_ref[...] = acc

    return pl.pallas_call(
        body,
        grid=(NPAD // BR,),
        in_specs=[
            pl.BlockSpec((NC, BR, D), lambda i: (0, i, 0)),
            pl.BlockSpec((BR, D), lambda i: (i, 0)),
            pl.BlockSpec((BR, 1), lambda i: (i, 0)),
            pl.BlockSpec((BR, EXPERTS), lambda i: (i, 0)),
            pl.BlockSpec((EXPERTS, D, D), lambda i: (0, 0, 0)),
            pl.BlockSpec((EXPERTS, D), lambda i: (0, 0)),
        ],
        out_specs=pl.BlockSpec((BR, D), lambda i: (i, 0)),
        out_shape=jax.ShapeDtypeStruct((NPAD, D), jnp.float32),
    )(agg_parts, xs, dsi, gate, W, b)


def kernel(x, edge_index, gate_features, W, b, Wg):
    ei = edge_index.astype(jnp.int32)
    src, dst = ei[0], ei[1]
    e = src.shape[0]
    cpt = -(-e // (NW * CHUNK))
    cpt = cpt + (cpt % 2)
    pad = NW * cpt * CHUNK - e
    pad_rows = N_NODES + (jnp.arange(pad, dtype=jnp.int32) % (NPAD - N_NODES))
    src_p = jnp.concatenate([src, pad_rows]).reshape(NW, cpt, CHUNK)
    dst_p = jnp.concatenate([dst, pad_rows]).reshape(NW, cpt, CHUNK)
    sd = jnp.stack([src_p, dst_p], axis=2)

    deg_parts = _deg_call(sd, cpt)
    xs, dsi, gate = _prep_call(deg_parts, x, gate_features, Wg)
    agg_parts = _agg_call(xs, sd, cpt)
    out = _combine_call(agg_parts, xs, dsi, gate, W, b)
    return out[:N_NODES]
